# Initial kernel scaffold; baseline (speedup 1.0000x reference)
import jax
import jax.numpy as jnp
from jax.experimental import pallas as pl


def kernel(x, edge_index, w1_1, b1_1, w2_1, w3_1, b3_1, w1_2, b1_2, w2_2, w3_2, b3_2):
    raise NotImplementedError("write your pallas kernel here")


# speedup vs baseline: 4.3736x; 4.3736x over previous
import functools

import jax
import jax.numpy as jnp
from jax import lax
from jax.experimental import pallas as pl
from jax.experimental.pallas import tpu as pltpu
from jax.experimental.pallas import tpu_sc as plsc

N = 50000
E = 800000
N_FEATS = 100
HIDDEN = 72
N_LABELS = 19

NC = 2
NS = 16
CH = 64
ZCH = 49

NP = 50176
EP = 802816
ER = EP // CH
STRIPE = NP // NS

W1 = 40
W2 = 24

BM = 512
GRID = NP // BM


def _sc_segsum_kernel(width, chunks_per_tile, edge_split, dual_table):
    mesh = plsc.VectorSubcoreMesh(core_axis_name="c", subcore_axis_name="s",
                                  num_cores=NC, num_subcores=NS)
    n_tab = 2 if dual_table else 1

    def body(*refs):
        tabs = refs[:n_tab]
        src_hbm, dst_hbm, zrows = refs[n_tab:n_tab + 3]
        out_hbm = refs[n_tab + 3]
        sidxb, didxb, rows, stage, sentv, accum = refs[n_tab + 4:n_tab + 10]

        cid = lax.axis_index("c")
        sid = lax.axis_index("s")

        pltpu.sync_copy(zrows, stage)

        def zero_step(k, _):
            pltpu.sync_copy(stage,
                            accum.at[pl.ds(sid * STRIPE + k * ZCH, ZCH)])
            return 0

        lax.fori_loop(0, STRIPE // ZCH, zero_step, 0)

        if edge_split:
            row0 = cid * (ER // NC) + sid * chunks_per_tile
        else:
            row0 = sid * chunks_per_tile

        plsc.subcore_barrier()

        seg = chunks_per_tile // 98
        for s in range(seg):
            def chunk_step(g, _, s=s):
                r = row0 + s * 98 + g
                pltpu.sync_copy(src_hbm.at[r], sidxb)
                pltpu.sync_copy(dst_hbm.at[r], didxb)
                if dual_table:
                    @pl.when(cid == 0)
                    def _():
                        pltpu.sync_copy(tabs[0].at[sidxb], rows)

                    @pl.when(cid == 1)
                    def _():
                        pltpu.sync_copy(tabs[1].at[sidxb], rows)
                else:
                    pltpu.sync_copy(tabs[0].at[sidxb], rows)
                pltpu.sync_copy(rows, accum.at[didxb], add=True)
                return 0

            lax.fori_loop(0, 98, chunk_step, 0)

        sent = NP - 64 + sid

        one_starts = sorted(set(list(range(0, width - 15, 16))
                                + [width - 16]))

        def fill_ones(j, _):
            for st in one_starts:
                rows[j, pl.ds(st, 16)] = jnp.full((16,), 1.0, jnp.float32)
            return 0

        lax.fori_loop(0, CH, fill_ones, 0)

        def fill_idx(k, _):
            didxb[pl.ds(k * 16, 16)] = jnp.full((16,), 1, jnp.int32) * sent
            return 0

        lax.fori_loop(0, CH // 16, fill_idx, 0)
        pltpu.sync_copy(rows, accum.at[didxb], add=True)

        def poll_body(k, done):
            @pl.when(done == 0)
            def _():
                pltpu.sync_copy(accum.at[pl.ds(sent, 1)], sentv)

            v = jnp.max(sentv[0, pl.ds(0, 16)])
            return jnp.where(v >= float(CH) - 0.5, 1, done)

        lax.fori_loop(0, 256, poll_body, 0)
        plsc.subcore_barrier()

        def out_step(k, _):
            pltpu.sync_copy(accum.at[pl.ds(sid * STRIPE + k * ZCH, ZCH)],
                            stage)
            pltpu.sync_copy(stage,
                            out_hbm.at[cid, pl.ds(sid * STRIPE + k * ZCH,
                                                  ZCH)])
            return 0

        lax.fori_loop(0, STRIPE // ZCH, out_step, 0)

    return pl.kernel(
        body,
        out_type=jax.ShapeDtypeStruct((NC, NP, width), jnp.float32),
        mesh=mesh,
        scratch_types=[
            pltpu.VMEM((CH,), jnp.int32),
            pltpu.VMEM((CH,), jnp.int32),
            pltpu.VMEM((CH, width), jnp.float32),
            pltpu.VMEM((ZCH, width), jnp.float32),
            pltpu.VMEM((1, width), jnp.float32),
            pltpu.VMEM_SHARED((NP, width), jnp.float32),
        ],
        compiler_params=pltpu.CompilerParams(use_tc_tiling_on_sc=False,
                                            needs_layout_passes=False),
    )


@functools.lru_cache(maxsize=None)
def _get_sc_kernels():
    l1 = _sc_segsum_kernel(W1, EP // (NS * CH), edge_split=False,
                           dual_table=True)
    l2 = _sc_segsum_kernel(W2, EP // (NC * NS * CH), edge_split=True,
                           dual_table=False)
    return l1, l2


def _mm1_body(x_ref, w_ref, bias_ref, t0_ref, t1_ref, b_ref, c_ref):
    i = pl.program_id(0)
    y = jnp.dot(x_ref[...], w_ref[...],
                preferred_element_type=jnp.float32) + bias_ref[...]
    row = i * BM + lax.broadcasted_iota(jnp.int32, (BM, 1), 0)
    mask = row < N
    t0_ref[...] = jnp.where(mask, y[:, 0:W1], 0.0)
    t1_ref[...] = jnp.where(mask, y[:, W1:2 * W1], 0.0)
    b_ref[...] = y[:, 80:152]
    c_ref[...] = y[:, 152:224]


def _mm1(xp, wc, bias):
    return pl.pallas_call(
        _mm1_body,
        grid=(GRID,),
        in_specs=[
            pl.BlockSpec((BM, 128), lambda i: (i, 0)),
            pl.BlockSpec((128, 224), lambda i: (0, 0)),
            pl.BlockSpec((1, 224), lambda i: (0, 0)),
        ],
        out_specs=[
            pl.BlockSpec((BM, W1), lambda i: (i, 0)),
            pl.BlockSpec((BM, W1), lambda i: (i, 0)),
            pl.BlockSpec((BM, HIDDEN), lambda i: (i, 0)),
            pl.BlockSpec((BM, HIDDEN), lambda i: (i, 0)),
        ],
        out_shape=[
            jax.ShapeDtypeStruct((NP, W1), jnp.float32),
            jax.ShapeDtypeStruct((NP, W1), jnp.float32),
            jax.ShapeDtypeStruct((NP, HIDDEN), jnp.float32),
            jax.ShapeDtypeStruct((NP, HIDDEN), jnp.float32),
        ],
    )(xp, wc, bias)


def _mm2_body(s_ref, b_ref, c_ref, w_ref, bias_ref, t2_ref, f2_ref):
    i = pl.program_id(0)
    s0 = s_ref[0]
    s1 = s_ref[1]
    seg = jnp.concatenate([s0[:, 0:36], s1[:, 0:36]], axis=1)
    deg = s1[:, 36:37]
    pre = seg + c_ref[...] - deg * b_ref[...]
    h = jnp.where(pre > 0, pre, jnp.exp(jnp.minimum(pre, 0.0)) - 1.0)
    y = jnp.dot(h, w_ref[...],
                preferred_element_type=jnp.float32) + bias_ref[...]
    row = i * BM + lax.broadcasted_iota(jnp.int32, (BM, 1), 0)
    mask = row < N
    t2_ref[...] = jnp.where(mask, y[:, 0:W2], 0.0)
    f2_ref[...] = y[:, 2 * W2:3 * W2] - deg * y[:, W2:2 * W2]


def _mm2(s1out, b1o, c1o, w2c, bias2):
    return pl.pallas_call(
        _mm2_body,
        grid=(GRID,),
        in_specs=[
            pl.BlockSpec((NC, BM, W1), lambda i: (0, i, 0)),
            pl.BlockSpec((BM, HIDDEN), lambda i: (i, 0)),
            pl.BlockSpec((BM, HIDDEN), lambda i: (i, 0)),
            pl.BlockSpec((HIDDEN, 3 * W2), lambda i: (0, 0)),
            pl.BlockSpec((1, 3 * W2), lambda i: (0, 0)),
        ],
        out_specs=[
            pl.BlockSpec((BM, W2), lambda i: (i, 0)),
            pl.BlockSpec((BM, W2), lambda i: (i, 0)),
        ],
        out_shape=[
            jax.ShapeDtypeStruct((NP, W2), jnp.float32),
            jax.ShapeDtypeStruct((NP, W2), jnp.float32),
        ],
    )(s1out, b1o, c1o, w2c, bias2)


def _final_body(s_ref, f2_ref, o_ref):
    z = s_ref[0] + s_ref[1] + f2_ref[...]
    col = lax.broadcasted_iota(jnp.int32, (BM, W2), 1)
    valid = col < N_LABELS
    zm = jnp.where(valid, z, -jnp.inf)
    m = jnp.max(zm, axis=1, keepdims=True)
    e = jnp.where(valid, jnp.exp(z - m), 0.0)
    lse = m + jnp.log(jnp.sum(e, axis=1, keepdims=True))
    o_ref[...] = (z - lse)[:, 0:N_LABELS]


def _final(s2out, f2):
    return pl.pallas_call(
        _final_body,
        grid=(GRID,),
        in_specs=[
            pl.BlockSpec((NC, BM, W2), lambda i: (0, i, 0)),
            pl.BlockSpec((BM, W2), lambda i: (i, 0)),
        ],
        out_specs=pl.BlockSpec((BM, N_LABELS), lambda i: (i, 0)),
        out_shape=jax.ShapeDtypeStruct((NP, N_LABELS), jnp.float32),
    )(s2out, f2)


def kernel(x, edge_index, w1_1, b1_1, w2_1, w3_1, b3_1,
           w1_2, b1_2, w2_2, w3_2, b3_2):
    f32 = jnp.float32

    xp = jnp.zeros((NP, 128), f32).at[:N, :N_FEATS].set(x.astype(f32))

    ei = edge_index.astype(jnp.int32)
    pad = jnp.full((2, EP - E), N, jnp.int32)
    eip = jnp.concatenate([ei, pad], axis=1)
    src2d = eip[0].reshape(ER, CH)
    dst2d = eip[1].reshape(ER, CH)

    wc1 = jnp.zeros((128, 224), f32)
    wc1 = wc1.at[:N_FEATS, 0:36].set(w1_1[:, 0:36])
    wc1 = wc1.at[:N_FEATS, 40:76].set(w1_1[:, 36:72])
    wc1 = wc1.at[:N_FEATS, 80:152].set(w2_1)
    wc1 = wc1.at[:N_FEATS, 152:224].set(w3_1)
    bias1 = jnp.zeros((1, 224), f32)
    bias1 = bias1.at[0, 0:36].set(b1_1[0:36])
    bias1 = bias1.at[0, 40:76].set(b1_1[36:72])
    bias1 = bias1.at[0, 76].set(1.0)
    bias1 = bias1.at[0, 152:224].set(b3_1)

    wc2 = jnp.zeros((HIDDEN, 3 * W2), f32)
    wc2 = wc2.at[:, 0:19].set(w1_2)
    wc2 = wc2.at[:, W2:W2 + 19].set(w2_2)
    wc2 = wc2.at[:, 2 * W2:2 * W2 + 19].set(w3_2)
    bias2 = jnp.zeros((1, 3 * W2), f32)
    bias2 = bias2.at[0, 0:19].set(b1_2)
    bias2 = bias2.at[0, 2 * W2:2 * W2 + 19].set(b3_2)

    z1 = jnp.zeros((ZCH, W1), f32)
    z2 = jnp.zeros((ZCH, W2), f32)

    sc_layer1, sc_layer2 = _get_sc_kernels()

    t0, t1, b1o, c1o = _mm1(xp, wc1, bias1)
    s1out = sc_layer1(t0, t1, src2d, dst2d, z1)
    t2, f2 = _mm2(s1out, b1o, c1o, wc2, bias2)

    s2out = sc_layer2(t2, src2d, dst2d, z2)
    out = _final(s2out, f2)

    return out[:N]
